# baseline (speedup 1.0000x reference)
import functools

import jax
import jax.numpy as jnp
from jax import lax
from jax.experimental import pallas as pl
from jax.experimental.pallas import tpu as pltpu
from jax.experimental.pallas import tpu_sc as plsc

_NUM_SEEDS = 16
_CVAR_ALPHA = 0.2
_CVAR_EPSILON = 0.3
_COST_SLM = 1.0
_COST_LLM = 50.0
_BRIER_WEIGHT = 1.0

_NC = 2
_NS = 16
_NW = _NC * _NS
_L = 16

_N = 1048576
_PW = _N // _NW
_C = 8192
_NCHUNK = _PW // _C
_U = 8
_NT = 2

_mesh = plsc.VectorSubcoreMesh(core_axis_name="c", subcore_axis_name="s")


@functools.partial(
    pl.kernel,
    mesh=_mesh,
    out_type=jax.ShapeDtypeStruct((_NW, 64), jnp.float32),
    compiler_params=pltpu.CompilerParams(needs_layout_passes=False),
    scratch_types=[
        pltpu.VMEM((_C,), jnp.float32),
        pltpu.VMEM((_C,), jnp.float32),
        pltpu.VMEM((_C,), jnp.float32),
        pltpu.VMEM((_C,), jnp.int32),
        pltpu.VMEM((_C,), jnp.float32),
        pltpu.VMEM((_C,), jnp.float32),
        pltpu.VMEM((_C,), jnp.float32),
        pltpu.VMEM((_C,), jnp.int32),
        pltpu.SemaphoreType.DMA,
        pltpu.SemaphoreType.DMA,
        pltpu.VMEM((_NUM_SEEDS * _L,), jnp.float32),
        pltpu.VMEM((_NUM_SEEDS * _L,), jnp.float32),
        pltpu.VMEM((_NUM_SEEDS * _L,), jnp.float32),
        pltpu.VMEM((_NUM_SEEDS * _L,), jnp.float32),
        pltpu.VMEM((1, 64), jnp.float32),
    ],
)
def _stage1(fp_hbm, lb_hbm, sc_hbm, sd_hbm, out_hbm,
            fp0, lb0, sc0, sd0, fp1, lb1, sc1, sd1, sem0, sem1,
            cnt_t0, cnt_t1, suc_t0, suc_t1, stage_v):
    cnt_t = (cnt_t0, cnt_t1)
    suc_t = (suc_t0, suc_t1)
    wid = lax.axis_index("s") * _NC + lax.axis_index("c")
    base = wid * _PW

    bufs = ((fp0, lb0, sc0, sd0), (fp1, lb1, sc1, sd1))
    sems = (sem0, sem1)

    def fire(k):
        slot = k % 2
        off = base + k * _C
        b = bufs[slot]
        sem = sems[slot]
        return [
            pltpu.async_copy(fp_hbm.at[pl.ds(off, _C)], b[0], sem),
            pltpu.async_copy(lb_hbm.at[pl.ds(off, _C)], b[1], sem),
            pltpu.async_copy(sc_hbm.at[pl.ds(off, _C)], b[2], sem),
            pltpu.async_copy(sd_hbm.at[pl.ds(off, _C)], b[3], sem),
        ]

    zeros16 = jnp.zeros((_L,), jnp.float32)
    ones16 = jnp.ones((_L,), jnp.float32)
    iota16 = lax.iota(jnp.int32, _L)
    for t in range(_NT):
        for j in range(_NUM_SEEDS):
            cnt_t[t][pl.ds(j * _L, _L)] = zeros16
            suc_t[t][pl.ds(j * _L, _L)] = zeros16

    acc_fp = [zeros16] * _U
    acc_sq = [zeros16] * _U
    handles = {0: fire(0)}
    for k in range(_NCHUNK):
        if k + 1 < _NCHUNK:
            handles[k + 1] = fire(k + 1)
        for h in handles.pop(k):
            h.wait()
        fp_v, lb_v, sc_v, sd_v = bufs[k % 2]

        def body(i, carry):
            afp, asq = carry
            afp, asq = list(afp), list(asq)
            for u in range(_U):
                j = i * (_L * _U) + u * _L
                fp = fp_v[pl.ds(j, _L)]
                lb = lb_v[pl.ds(j, _L)]
                sc = sc_v[pl.ds(j, _L)]
                sd = sd_v[pl.ds(j, _L)]
                idx = sd * _L + iota16
                plsc.addupdate_scatter(cnt_t[u % _NT], [idx], ones16)
                plsc.addupdate_scatter(suc_t[u % _NT], [idx], sc)
                d = fp - lb
                afp[u] = afp[u] + fp
                asq[u] = asq[u] + d * d
            return tuple(afp), tuple(asq)

        acc_fp, acc_sq = lax.fori_loop(
            0, _C // (_L * _U), body, (tuple(acc_fp), tuple(acc_sq)))

    acc_fp = sum(acc_fp[1:], acc_fp[0])
    acc_sq = sum(acc_sq[1:], acc_sq[0])

    cnt = zeros16
    suc = zeros16
    for b in range(_NUM_SEEDS):
        sc_b = jnp.float32(0)
        ss_b = jnp.float32(0)
        for t in range(_NT):
            sc_b = sc_b + jnp.sum(cnt_t[t][pl.ds(b * _L, _L)])
            ss_b = ss_b + jnp.sum(suc_t[t][pl.ds(b * _L, _L)])
        cnt = jnp.where(iota16 == b, sc_b, cnt)
        suc = jnp.where(iota16 == b, ss_b, suc)

    stage_v[0, pl.ds(0, _L)] = cnt
    stage_v[0, pl.ds(16, _L)] = suc
    stage_v[0, pl.ds(32, _L)] = acc_fp
    stage_v[0, pl.ds(48, _L)] = acc_sq
    pltpu.sync_copy(stage_v, out_hbm.at[pl.ds(wid, 1)])


def _finalize_body(parts_ref, lam_ref, out_ref):
    parts = parts_ref[...]
    tot = jnp.sum(parts, axis=0, keepdims=True)
    cnt = tot[:, 0:16]
    suc = tot[:, 16:32]
    sum_fp = jnp.sum(tot[:, 32:48])
    sum_sq = jnp.sum(tot[:, 48:64])
    lam = lam_ref[0, 0]

    sf = (cnt - suc) / jnp.maximum(cnt, 1.0)
    idx = lax.broadcasted_iota(jnp.int32, (1, _NUM_SEEDS), 1)

    rank = jnp.zeros((1, _NUM_SEEDS), jnp.int32)
    for i in range(_NUM_SEEDS):
        sfi = jnp.sum(jnp.where(idx == i, sf, 0.0))
        gt = (sfi > sf) | ((sfi == sf) & (i < idx))
        rank = rank + gt.astype(jnp.int32)
    kk = max(1, int(_NUM_SEEDS * _CVAR_ALPHA))
    rob = jnp.sum(jnp.where(rank < kk, sf, 0.0)) * (1.0 / kk)

    inv_n = 1.0 / _N
    cost = (_COST_LLM - _COST_SLM) * (sum_fp * inv_n) + _COST_SLM
    brier = sum_sq * inv_n
    cv = rob - _CVAR_EPSILON
    lagr = lam * cv
    total = cost + lagr + _BRIER_WEIGHT * brier
    dual = -lam * cv

    o = lax.broadcasted_iota(jnp.int32, (1, 8), 1)
    out_ref[...] = jnp.where(
        o == 0, total,
        jnp.where(o == 1, cost,
                  jnp.where(o == 2, rob,
                            jnp.where(o == 3, brier,
                                      jnp.where(o == 4, lagr,
                                                jnp.where(o == 5, dual,
                                                          jnp.where(o == 6, cv, 0.0)))))))


_finalize = pl.pallas_call(
    _finalize_body,
    out_shape=jax.ShapeDtypeStruct((1, 8), jnp.float32),
)


def kernel(fallback_probs, labels, success, perturbation_seeds, lagrange_multiplier):
    seeds32 = perturbation_seeds.astype(jnp.int32)
    parts = _stage1(fallback_probs, labels, success, seeds32)
    lam2 = lagrange_multiplier.astype(jnp.float32).reshape(1, 1)
    o = _finalize(parts, lam2)
    return (
        o[0, 0],
        o[0, 1],
        o[0, 2],
        o[0, 3],
        o[0, 4],
        o[0, 5],
        o[0, 6],
    )


# speedup vs baseline: 1.0167x; 1.0167x over previous
import functools

import jax
import jax.numpy as jnp
from jax import lax
from jax.experimental import pallas as pl
from jax.experimental.pallas import tpu as pltpu
from jax.experimental.pallas import tpu_sc as plsc

_NUM_SEEDS = 16
_CVAR_ALPHA = 0.2
_CVAR_EPSILON = 0.3
_COST_SLM = 1.0
_COST_LLM = 50.0
_BRIER_WEIGHT = 1.0

_NC = 2
_NS = 16
_NW = _NC * _NS
_L = 16

_N = 1048576
_PW = _N // _NW
_C = 4096
_NCHUNK = _PW // _C
_U = 4
_NT = 4

_mesh = plsc.VectorSubcoreMesh(core_axis_name="c", subcore_axis_name="s")


@functools.partial(
    pl.kernel,
    mesh=_mesh,
    out_type=jax.ShapeDtypeStruct((_NW, 64), jnp.float32),
    compiler_params=pltpu.CompilerParams(needs_layout_passes=False),
    scratch_types=[
        pltpu.VMEM((_C,), jnp.float32),
        pltpu.VMEM((_C,), jnp.float32),
        pltpu.VMEM((_C,), jnp.float32),
        pltpu.VMEM((_C,), jnp.int32),
        pltpu.VMEM((_C,), jnp.float32),
        pltpu.VMEM((_C,), jnp.float32),
        pltpu.VMEM((_C,), jnp.float32),
        pltpu.VMEM((_C,), jnp.int32),
        pltpu.SemaphoreType.DMA,
        pltpu.SemaphoreType.DMA,
        pltpu.VMEM((_NUM_SEEDS * _L,), jnp.float32),
        pltpu.VMEM((_NUM_SEEDS * _L,), jnp.float32),
        pltpu.VMEM((_NUM_SEEDS * _L,), jnp.float32),
        pltpu.VMEM((_NUM_SEEDS * _L,), jnp.float32),
        pltpu.VMEM((_NUM_SEEDS * _L,), jnp.float32),
        pltpu.VMEM((_NUM_SEEDS * _L,), jnp.float32),
        pltpu.VMEM((_NUM_SEEDS * _L,), jnp.float32),
        pltpu.VMEM((_NUM_SEEDS * _L,), jnp.float32),
        pltpu.VMEM((1, 64), jnp.float32),
    ],
)
def _stage1(fp_hbm, lb_hbm, sc_hbm, sd_hbm, out_hbm,
            fp0, lb0, sc0, sd0, fp1, lb1, sc1, sd1, sem0, sem1,
            cnt_t0, cnt_t1, cnt_t2, cnt_t3, suc_t0, suc_t1, suc_t2, suc_t3, stage_v):
    cnt_t = (cnt_t0, cnt_t1, cnt_t2, cnt_t3)
    suc_t = (suc_t0, suc_t1, suc_t2, suc_t3)
    wid = lax.axis_index("s") * _NC + lax.axis_index("c")
    base = wid * _PW

    bufs = ((fp0, lb0, sc0, sd0), (fp1, lb1, sc1, sd1))
    sems = (sem0, sem1)

    def fire(k):
        slot = k % 2
        off = base + k * _C
        b = bufs[slot]
        sem = sems[slot]
        return [
            pltpu.async_copy(fp_hbm.at[pl.ds(off, _C)], b[0], sem),
            pltpu.async_copy(lb_hbm.at[pl.ds(off, _C)], b[1], sem),
            pltpu.async_copy(sc_hbm.at[pl.ds(off, _C)], b[2], sem),
            pltpu.async_copy(sd_hbm.at[pl.ds(off, _C)], b[3], sem),
        ]

    zeros16 = jnp.zeros((_L,), jnp.float32)
    ones16 = jnp.ones((_L,), jnp.float32)
    iota16 = lax.iota(jnp.int32, _L)
    for t in range(_NT):
        for j in range(_NUM_SEEDS):
            cnt_t[t][pl.ds(j * _L, _L)] = zeros16
            suc_t[t][pl.ds(j * _L, _L)] = zeros16

    acc_fp = [zeros16] * _U
    acc_sq = [zeros16] * _U
    handles = {0: fire(0)}
    for k in range(_NCHUNK):
        if k + 1 < _NCHUNK:
            handles[k + 1] = fire(k + 1)
        for h in handles.pop(k):
            h.wait()
        fp_v, lb_v, sc_v, sd_v = bufs[k % 2]

        def body(i, carry):
            afp, asq = carry
            afp, asq = list(afp), list(asq)
            for u in range(_U):
                j = i * (_L * _U) + u * _L
                fp = fp_v[pl.ds(j, _L)]
                lb = lb_v[pl.ds(j, _L)]
                sc = sc_v[pl.ds(j, _L)]
                sd = sd_v[pl.ds(j, _L)]
                idx = sd * _L + iota16
                plsc.addupdate_scatter(cnt_t[u % _NT], [idx], ones16)
                plsc.addupdate_scatter(suc_t[u % _NT], [idx], sc)
                d = fp - lb
                afp[u] = afp[u] + fp
                asq[u] = asq[u] + d * d
            return tuple(afp), tuple(asq)

        acc_fp, acc_sq = lax.fori_loop(
            0, _C // (_L * _U), body, (tuple(acc_fp), tuple(acc_sq)))

    acc_fp = sum(acc_fp[1:], acc_fp[0])
    acc_sq = sum(acc_sq[1:], acc_sq[0])

    cnt = zeros16
    suc = zeros16
    for b in range(_NUM_SEEDS):
        sc_b = jnp.float32(0)
        ss_b = jnp.float32(0)
        for t in range(_NT):
            sc_b = sc_b + jnp.sum(cnt_t[t][pl.ds(b * _L, _L)])
            ss_b = ss_b + jnp.sum(suc_t[t][pl.ds(b * _L, _L)])
        cnt = jnp.where(iota16 == b, sc_b, cnt)
        suc = jnp.where(iota16 == b, ss_b, suc)

    stage_v[0, pl.ds(0, _L)] = cnt
    stage_v[0, pl.ds(16, _L)] = suc
    stage_v[0, pl.ds(32, _L)] = acc_fp
    stage_v[0, pl.ds(48, _L)] = acc_sq
    pltpu.sync_copy(stage_v, out_hbm.at[pl.ds(wid, 1)])


def _finalize_body(parts_ref, lam_ref, out_ref):
    parts = parts_ref[...]
    tot = jnp.sum(parts, axis=0, keepdims=True)
    cnt = tot[:, 0:16]
    suc = tot[:, 16:32]
    sum_fp = jnp.sum(tot[:, 32:48])
    sum_sq = jnp.sum(tot[:, 48:64])
    lam = lam_ref[0, 0]

    sf = (cnt - suc) / jnp.maximum(cnt, 1.0)
    idx = lax.broadcasted_iota(jnp.int32, (1, _NUM_SEEDS), 1)

    rank = jnp.zeros((1, _NUM_SEEDS), jnp.int32)
    for i in range(_NUM_SEEDS):
        sfi = jnp.sum(jnp.where(idx == i, sf, 0.0))
        gt = (sfi > sf) | ((sfi == sf) & (i < idx))
        rank = rank + gt.astype(jnp.int32)
    kk = max(1, int(_NUM_SEEDS * _CVAR_ALPHA))
    rob = jnp.sum(jnp.where(rank < kk, sf, 0.0)) * (1.0 / kk)

    inv_n = 1.0 / _N
    cost = (_COST_LLM - _COST_SLM) * (sum_fp * inv_n) + _COST_SLM
    brier = sum_sq * inv_n
    cv = rob - _CVAR_EPSILON
    lagr = lam * cv
    total = cost + lagr + _BRIER_WEIGHT * brier
    dual = -lam * cv

    o = lax.broadcasted_iota(jnp.int32, (1, 8), 1)
    out_ref[...] = jnp.where(
        o == 0, total,
        jnp.where(o == 1, cost,
                  jnp.where(o == 2, rob,
                            jnp.where(o == 3, brier,
                                      jnp.where(o == 4, lagr,
                                                jnp.where(o == 5, dual,
                                                          jnp.where(o == 6, cv, 0.0)))))))


_finalize = pl.pallas_call(
    _finalize_body,
    out_shape=jax.ShapeDtypeStruct((1, 8), jnp.float32),
)


def kernel(fallback_probs, labels, success, perturbation_seeds, lagrange_multiplier):
    seeds32 = perturbation_seeds.astype(jnp.int32)
    parts = _stage1(fallback_probs, labels, success, seeds32)
    lam2 = lagrange_multiplier.astype(jnp.float32).reshape(1, 1)
    o = _finalize(parts, lam2)
    return (
        o[0, 0],
        o[0, 1],
        o[0, 2],
        o[0, 3],
        o[0, 4],
        o[0, 5],
        o[0, 6],
    )


# speedup vs baseline: 1.0241x; 1.0072x over previous
import functools

import jax
import jax.numpy as jnp
from jax import lax
from jax.experimental import pallas as pl
from jax.experimental.pallas import tpu as pltpu
from jax.experimental.pallas import tpu_sc as plsc

_NUM_SEEDS = 16
_CVAR_ALPHA = 0.2
_CVAR_EPSILON = 0.3
_COST_SLM = 1.0
_COST_LLM = 50.0
_BRIER_WEIGHT = 1.0

_NC = 2
_NS = 16
_NW = _NC * _NS
_L = 16

_N = 1048576
_PW = _N // _NW
_C = 4096
_NCHUNK = _PW // _C
_U = 4
_NT = 2

_mesh = plsc.VectorSubcoreMesh(core_axis_name="c", subcore_axis_name="s")


@functools.partial(
    pl.kernel,
    mesh=_mesh,
    out_type=jax.ShapeDtypeStruct((_NW, 32), jnp.float32),
    compiler_params=pltpu.CompilerParams(needs_layout_passes=False),
    scratch_types=[
        pltpu.VMEM((_C,), jnp.float32),
        pltpu.VMEM((_C,), jnp.int32),
        pltpu.VMEM((_C,), jnp.float32),
        pltpu.VMEM((_C,), jnp.int32),
        pltpu.SemaphoreType.DMA,
        pltpu.SemaphoreType.DMA,
        pltpu.VMEM((_NUM_SEEDS * _L,), jnp.float32),
        pltpu.VMEM((_NUM_SEEDS * _L,), jnp.float32),
        pltpu.VMEM((_NUM_SEEDS * _L,), jnp.float32),
        pltpu.VMEM((_NUM_SEEDS * _L,), jnp.float32),
        pltpu.VMEM((1, 32), jnp.float32),
    ],
)
def _seed_hist(sc_hbm, sd_hbm, out_hbm,
               sc0, sd0, sc1, sd1, sem0, sem1,
               cnt_t0, cnt_t1, suc_t0, suc_t1, stage_v):
    cnt_t = (cnt_t0, cnt_t1)
    suc_t = (suc_t0, suc_t1)
    wid = lax.axis_index("s") * _NC + lax.axis_index("c")
    base = wid * _PW

    bufs = ((sc0, sd0), (sc1, sd1))
    sems = (sem0, sem1)

    def fire(k):
        slot = k % 2
        off = base + k * _C
        b = bufs[slot]
        sem = sems[slot]
        return [
            pltpu.async_copy(sc_hbm.at[pl.ds(off, _C)], b[0], sem),
            pltpu.async_copy(sd_hbm.at[pl.ds(off, _C)], b[1], sem),
        ]

    zeros16 = jnp.zeros((_L,), jnp.float32)
    ones16 = jnp.ones((_L,), jnp.float32)
    iota16 = lax.iota(jnp.int32, _L)
    for t in range(_NT):
        for j in range(_NUM_SEEDS):
            cnt_t[t][pl.ds(j * _L, _L)] = zeros16
            suc_t[t][pl.ds(j * _L, _L)] = zeros16

    handles = {0: fire(0)}
    for k in range(_NCHUNK):
        if k + 1 < _NCHUNK:
            handles[k + 1] = fire(k + 1)
        for h in handles.pop(k):
            h.wait()
        sc_v, sd_v = bufs[k % 2]

        def body(i, _):
            for u in range(_U):
                j = i * (_L * _U) + u * _L
                sc = sc_v[pl.ds(j, _L)]
                sd = sd_v[pl.ds(j, _L)]
                idx = sd * _L + iota16
                plsc.addupdate_scatter(cnt_t[u % _NT], [idx], ones16)
                plsc.addupdate_scatter(suc_t[u % _NT], [idx], sc)
            return 0

        lax.fori_loop(0, _C // (_L * _U), body, 0)

    cnt = zeros16
    suc = zeros16
    for b in range(_NUM_SEEDS):
        sc_b = jnp.float32(0)
        ss_b = jnp.float32(0)
        for t in range(_NT):
            sc_b = sc_b + jnp.sum(cnt_t[t][pl.ds(b * _L, _L)])
            ss_b = ss_b + jnp.sum(suc_t[t][pl.ds(b * _L, _L)])
        cnt = jnp.where(iota16 == b, sc_b, cnt)
        suc = jnp.where(iota16 == b, ss_b, suc)

    stage_v[0, pl.ds(0, _L)] = cnt
    stage_v[0, pl.ds(16, _L)] = suc
    pltpu.sync_copy(stage_v, out_hbm.at[pl.ds(wid, 1)])


def _dense_body(fp_ref, lb_ref, out_ref):
    fp = fp_ref[...]
    lb = lb_ref[...]
    d = fp - lb
    o = lax.broadcasted_iota(jnp.int32, (1, 8), 1)
    out_ref[...] = jnp.where(o == 0, jnp.sum(fp),
                             jnp.where(o == 1, jnp.sum(d * d), 0.0))


_dense = pl.pallas_call(
    _dense_body,
    out_shape=jax.ShapeDtypeStruct((1, 8), jnp.float32),
)


def _finalize_body(parts_ref, dense_ref, lam_ref, out_ref):
    parts = parts_ref[...]
    tot = jnp.sum(parts, axis=0, keepdims=True)
    cnt = tot[:, 0:16]
    suc = tot[:, 16:32]
    sum_fp = dense_ref[0, 0]
    sum_sq = dense_ref[0, 1]
    lam = lam_ref[0, 0]

    sf = (cnt - suc) / jnp.maximum(cnt, 1.0)
    idx = lax.broadcasted_iota(jnp.int32, (1, _NUM_SEEDS), 1)

    rank = jnp.zeros((1, _NUM_SEEDS), jnp.int32)
    for i in range(_NUM_SEEDS):
        sfi = jnp.sum(jnp.where(idx == i, sf, 0.0))
        gt = (sfi > sf) | ((sfi == sf) & (i < idx))
        rank = rank + gt.astype(jnp.int32)
    kk = max(1, int(_NUM_SEEDS * _CVAR_ALPHA))
    rob = jnp.sum(jnp.where(rank < kk, sf, 0.0)) * (1.0 / kk)

    inv_n = 1.0 / _N
    cost = (_COST_LLM - _COST_SLM) * (sum_fp * inv_n) + _COST_SLM
    brier = sum_sq * inv_n
    cv = rob - _CVAR_EPSILON
    lagr = lam * cv
    total = cost + lagr + _BRIER_WEIGHT * brier
    dual = -lam * cv

    o = lax.broadcasted_iota(jnp.int32, (1, 8), 1)
    out_ref[...] = jnp.where(
        o == 0, total,
        jnp.where(o == 1, cost,
                  jnp.where(o == 2, rob,
                            jnp.where(o == 3, brier,
                                      jnp.where(o == 4, lagr,
                                                jnp.where(o == 5, dual,
                                                          jnp.where(o == 6, cv, 0.0)))))))


_finalize = pl.pallas_call(
    _finalize_body,
    out_shape=jax.ShapeDtypeStruct((1, 8), jnp.float32),
)


def kernel(fallback_probs, labels, success, perturbation_seeds, lagrange_multiplier):
    seeds32 = perturbation_seeds.astype(jnp.int32)
    parts = _seed_hist(success, seeds32)
    fp2 = fallback_probs.reshape(_N // 128, 128)
    lb2 = labels.reshape(_N // 128, 128)
    dense = _dense(fp2, lb2)
    lam2 = lagrange_multiplier.astype(jnp.float32).reshape(1, 1)
    o = _finalize(parts, dense, lam2)
    return (
        o[0, 0],
        o[0, 1],
        o[0, 2],
        o[0, 3],
        o[0, 4],
        o[0, 5],
        o[0, 6],
    )


# speedup vs baseline: 1.0373x; 1.0130x over previous
import functools

import jax
import jax.numpy as jnp
from jax import lax
from jax.experimental import pallas as pl
from jax.experimental.pallas import tpu as pltpu
from jax.experimental.pallas import tpu_sc as plsc

_NUM_SEEDS = 16
_CVAR_ALPHA = 0.2
_CVAR_EPSILON = 0.3
_COST_SLM = 1.0
_COST_LLM = 50.0
_BRIER_WEIGHT = 1.0

_NC = 2
_NS = 16
_NW = _NC * _NS
_L = 16

_N = 1048576
_PW = _N // _NW
_C = 4096
_NCHUNK = _PW // _C
_U = 4
_NT = 2

_mesh = plsc.VectorSubcoreMesh(core_axis_name="c", subcore_axis_name="s")


@functools.partial(
    pl.kernel,
    mesh=_mesh,
    out_type=jax.ShapeDtypeStruct((_NW, 16), jnp.float32),
    compiler_params=pltpu.CompilerParams(needs_layout_passes=False),
    scratch_types=[
        pltpu.VMEM((_C,), jnp.float32),
        pltpu.VMEM((_C,), jnp.int32),
        pltpu.VMEM((_C,), jnp.float32),
        pltpu.VMEM((_C,), jnp.int32),
        pltpu.SemaphoreType.DMA,
        pltpu.SemaphoreType.DMA,
        pltpu.VMEM((_NUM_SEEDS * _L,), jnp.float32),
        pltpu.VMEM((_NUM_SEEDS * _L,), jnp.float32),
        pltpu.VMEM((1, 16), jnp.float32),
    ],
)
def _seed_hist(sc_hbm, sd_hbm, out_hbm,
               sc0, sd0, sc1, sd1, sem0, sem1,
               suc_t0, suc_t1, stage_v):
    suc_t = (suc_t0, suc_t1)
    wid = lax.axis_index("s") * _NC + lax.axis_index("c")
    base = wid * _PW

    bufs = ((sc0, sd0), (sc1, sd1))
    sems = (sem0, sem1)

    def fire(k):
        slot = k % 2
        off = base + k * _C
        b = bufs[slot]
        sem = sems[slot]
        return [
            pltpu.async_copy(sc_hbm.at[pl.ds(off, _C)], b[0], sem),
            pltpu.async_copy(sd_hbm.at[pl.ds(off, _C)], b[1], sem),
        ]

    zeros16 = jnp.zeros((_L,), jnp.float32)
    iota16 = lax.iota(jnp.int32, _L)
    for t in range(_NT):
        for j in range(_NUM_SEEDS):
            suc_t[t][pl.ds(j * _L, _L)] = zeros16

    handles = {0: fire(0)}
    for k in range(_NCHUNK):
        if k + 1 < _NCHUNK:
            handles[k + 1] = fire(k + 1)
        for h in handles.pop(k):
            h.wait()
        sc_v, sd_v = bufs[k % 2]

        def body(i, _):
            for u in range(_U):
                j = i * (_L * _U) + u * _L
                sc = sc_v[pl.ds(j, _L)]
                sd = sd_v[pl.ds(j, _L)]
                idx = sd * _L + iota16
                plsc.addupdate_scatter(suc_t[u % _NT], [idx], sc)
            return 0

        lax.fori_loop(0, _C // (_L * _U), body, 0)

    suc = zeros16
    for b in range(_NUM_SEEDS):
        ss_b = jnp.float32(0)
        for t in range(_NT):
            ss_b = ss_b + jnp.sum(suc_t[t][pl.ds(b * _L, _L)])
        suc = jnp.where(iota16 == b, ss_b, suc)

    stage_v[0, :] = suc
    pltpu.sync_copy(stage_v, out_hbm.at[pl.ds(wid, 1)])


def _dense_body(fp_ref, lb_ref, sd_ref, out_ref):
    fp = fp_ref[...]
    lb = lb_ref[...]
    sd = sd_ref[...]
    d = fp - lb
    o = lax.broadcasted_iota(jnp.int32, (1, 32), 1)
    out = jnp.where(o == 0, jnp.sum(fp),
                    jnp.where(o == 1, jnp.sum(d * d), 0.0))
    for b in range(_NUM_SEEDS):
        cnt_b = jnp.sum((sd == b).astype(jnp.float32))
        out = jnp.where(o == 16 + b, cnt_b, out)
    out_ref[...] = out


_dense = pl.pallas_call(
    _dense_body,
    out_shape=jax.ShapeDtypeStruct((1, 32), jnp.float32),
)


def _finalize_body(parts_ref, dense_ref, lam_ref, out_ref):
    parts = parts_ref[...]
    suc = jnp.sum(parts, axis=0, keepdims=True)
    cnt = dense_ref[:, 16:32]
    sum_fp = dense_ref[0, 0]
    sum_sq = dense_ref[0, 1]
    lam = lam_ref[0, 0]

    sf = (cnt - suc) / jnp.maximum(cnt, 1.0)
    idx = lax.broadcasted_iota(jnp.int32, (1, _NUM_SEEDS), 1)

    rank = jnp.zeros((1, _NUM_SEEDS), jnp.int32)
    for i in range(_NUM_SEEDS):
        sfi = jnp.sum(jnp.where(idx == i, sf, 0.0))
        gt = (sfi > sf) | ((sfi == sf) & (i < idx))
        rank = rank + gt.astype(jnp.int32)
    kk = max(1, int(_NUM_SEEDS * _CVAR_ALPHA))
    rob = jnp.sum(jnp.where(rank < kk, sf, 0.0)) * (1.0 / kk)

    inv_n = 1.0 / _N
    cost = (_COST_LLM - _COST_SLM) * (sum_fp * inv_n) + _COST_SLM
    brier = sum_sq * inv_n
    cv = rob - _CVAR_EPSILON
    lagr = lam * cv
    total = cost + lagr + _BRIER_WEIGHT * brier
    dual = -lam * cv

    o = lax.broadcasted_iota(jnp.int32, (1, 8), 1)
    out_ref[...] = jnp.where(
        o == 0, total,
        jnp.where(o == 1, cost,
                  jnp.where(o == 2, rob,
                            jnp.where(o == 3, brier,
                                      jnp.where(o == 4, lagr,
                                                jnp.where(o == 5, dual,
                                                          jnp.where(o == 6, cv, 0.0)))))))


_finalize = pl.pallas_call(
    _finalize_body,
    out_shape=jax.ShapeDtypeStruct((1, 8), jnp.float32),
)


def kernel(fallback_probs, labels, success, perturbation_seeds, lagrange_multiplier):
    seeds32 = perturbation_seeds.astype(jnp.int32)
    parts = _seed_hist(success, seeds32)
    fp2 = fallback_probs.reshape(_N // 128, 128)
    lb2 = labels.reshape(_N // 128, 128)
    sd2 = seeds32.reshape(_N // 128, 128)
    dense = _dense(fp2, lb2, sd2)
    lam2 = lagrange_multiplier.astype(jnp.float32).reshape(1, 1)
    o = _finalize(parts, dense, lam2)
    return (
        o[0, 0],
        o[0, 1],
        o[0, 2],
        o[0, 3],
        o[0, 4],
        o[0, 5],
        o[0, 6],
    )


# speedup vs baseline: 1.2944x; 1.2478x over previous
import functools

import jax
import jax.numpy as jnp
from jax import lax
from jax.experimental import pallas as pl
from jax.experimental.pallas import tpu as pltpu
from jax.experimental.pallas import tpu_sc as plsc

_NUM_SEEDS = 16
_CVAR_ALPHA = 0.2
_CVAR_EPSILON = 0.3
_COST_SLM = 1.0
_COST_LLM = 50.0
_BRIER_WEIGHT = 1.0

_NC = 2
_NS = 16
_NW = _NC * _NS
_L = 16

_N = 1048576
_PW = _N // _NW
_C = 4096
_NCHUNK = _PW // _C
_U = 4
_NT = 2

_mesh = plsc.VectorSubcoreMesh(core_axis_name="c", subcore_axis_name="s")


@functools.partial(
    pl.kernel,
    mesh=_mesh,
    out_type=jax.ShapeDtypeStruct((_NW, 16), jnp.float32),
    compiler_params=pltpu.CompilerParams(needs_layout_passes=False),
    scratch_types=[
        pltpu.VMEM((_C,), jnp.float32),
        pltpu.VMEM((_C,), jnp.int32),
        pltpu.VMEM((_C,), jnp.float32),
        pltpu.VMEM((_C,), jnp.int32),
        pltpu.SemaphoreType.DMA,
        pltpu.SemaphoreType.DMA,
        pltpu.VMEM((_NUM_SEEDS * _L,), jnp.float32),
        pltpu.VMEM((_NUM_SEEDS * _L,), jnp.float32),
        pltpu.VMEM((1, 16), jnp.float32),
    ],
)
def _seed_hist(sc_hbm, sd_hbm, out_hbm,
               sc0, sd0, sc1, sd1, sem0, sem1,
               suc_t0, suc_t1, stage_v):
    suc_t = (suc_t0, suc_t1)
    wid = lax.axis_index("s") * _NC + lax.axis_index("c")
    base = wid * _PW

    bufs = ((sc0, sd0), (sc1, sd1))
    sems = (sem0, sem1)

    def fire(k):
        slot = k % 2
        off = base + k * _C
        b = bufs[slot]
        sem = sems[slot]
        return [
            pltpu.async_copy(sc_hbm.at[pl.ds(off, _C)], b[0], sem),
            pltpu.async_copy(sd_hbm.at[pl.ds(off, _C)], b[1], sem),
        ]

    zeros16 = jnp.zeros((_L,), jnp.float32)
    iota16 = lax.iota(jnp.int32, _L)
    for t in range(_NT):
        for j in range(_NUM_SEEDS):
            suc_t[t][pl.ds(j * _L, _L)] = zeros16

    handles = {0: fire(0)}
    for k in range(_NCHUNK):
        if k + 1 < _NCHUNK:
            handles[k + 1] = fire(k + 1)
        for h in handles.pop(k):
            h.wait()
        sc_v, sd_v = bufs[k % 2]
        tab = suc_t[k % _NT]

        @plsc.parallel_loop(0, _C // _L, unroll=_U)
        def _(i):
            j = i * _L
            sc = sc_v[pl.ds(j, _L)]
            sd = sd_v[pl.ds(j, _L)]
            idx = sd * _L + iota16
            plsc.addupdate_scatter(tab, [idx], sc)

    suc = zeros16
    for b in range(_NUM_SEEDS):
        ss_b = jnp.float32(0)
        for t in range(_NT):
            ss_b = ss_b + jnp.sum(suc_t[t][pl.ds(b * _L, _L)])
        suc = jnp.where(iota16 == b, ss_b, suc)

    stage_v[0, :] = suc
    pltpu.sync_copy(stage_v, out_hbm.at[pl.ds(wid, 1)])


def _dense_body(fp_ref, lb_ref, sd_ref, out_ref):
    fp = fp_ref[...]
    lb = lb_ref[...]
    sd = sd_ref[...]
    d = fp - lb
    o = lax.broadcasted_iota(jnp.int32, (1, 32), 1)
    out = jnp.where(o == 0, jnp.sum(fp),
                    jnp.where(o == 1, jnp.sum(d * d), 0.0))
    for b in range(_NUM_SEEDS):
        cnt_b = jnp.sum((sd == b).astype(jnp.float32))
        out = jnp.where(o == 16 + b, cnt_b, out)
    out_ref[...] = out


_dense = pl.pallas_call(
    _dense_body,
    out_shape=jax.ShapeDtypeStruct((1, 32), jnp.float32),
)


def _finalize_body(parts_ref, dense_ref, lam_ref, out_ref):
    parts = parts_ref[...]
    suc = jnp.sum(parts, axis=0, keepdims=True)
    cnt = dense_ref[:, 16:32]
    sum_fp = dense_ref[0, 0]
    sum_sq = dense_ref[0, 1]
    lam = lam_ref[0, 0]

    sf = (cnt - suc) / jnp.maximum(cnt, 1.0)
    idx = lax.broadcasted_iota(jnp.int32, (1, _NUM_SEEDS), 1)

    rank = jnp.zeros((1, _NUM_SEEDS), jnp.int32)
    for i in range(_NUM_SEEDS):
        sfi = jnp.sum(jnp.where(idx == i, sf, 0.0))
        gt = (sfi > sf) | ((sfi == sf) & (i < idx))
        rank = rank + gt.astype(jnp.int32)
    kk = max(1, int(_NUM_SEEDS * _CVAR_ALPHA))
    rob = jnp.sum(jnp.where(rank < kk, sf, 0.0)) * (1.0 / kk)

    inv_n = 1.0 / _N
    cost = (_COST_LLM - _COST_SLM) * (sum_fp * inv_n) + _COST_SLM
    brier = sum_sq * inv_n
    cv = rob - _CVAR_EPSILON
    lagr = lam * cv
    total = cost + lagr + _BRIER_WEIGHT * brier
    dual = -lam * cv

    o = lax.broadcasted_iota(jnp.int32, (1, 8), 1)
    out_ref[...] = jnp.where(
        o == 0, total,
        jnp.where(o == 1, cost,
                  jnp.where(o == 2, rob,
                            jnp.where(o == 3, brier,
                                      jnp.where(o == 4, lagr,
                                                jnp.where(o == 5, dual,
                                                          jnp.where(o == 6, cv, 0.0)))))))


_finalize = pl.pallas_call(
    _finalize_body,
    out_shape=jax.ShapeDtypeStruct((1, 8), jnp.float32),
)


def kernel(fallback_probs, labels, success, perturbation_seeds, lagrange_multiplier):
    seeds32 = perturbation_seeds.astype(jnp.int32)
    parts = _seed_hist(success, seeds32)
    fp2 = fallback_probs.reshape(_N // 128, 128)
    lb2 = labels.reshape(_N // 128, 128)
    sd2 = seeds32.reshape(_N // 128, 128)
    dense = _dense(fp2, lb2, sd2)
    lam2 = lagrange_multiplier.astype(jnp.float32).reshape(1, 1)
    o = _finalize(parts, dense, lam2)
    return (
        o[0, 0],
        o[0, 1],
        o[0, 2],
        o[0, 3],
        o[0, 4],
        o[0, 5],
        o[0, 6],
    )


# speedup vs baseline: 1.3279x; 1.0259x over previous
import functools

import jax
import jax.numpy as jnp
from jax import lax
from jax.experimental import pallas as pl
from jax.experimental.pallas import tpu as pltpu
from jax.experimental.pallas import tpu_sc as plsc

_NUM_SEEDS = 16
_CVAR_ALPHA = 0.2
_CVAR_EPSILON = 0.3
_COST_SLM = 1.0
_COST_LLM = 50.0
_BRIER_WEIGHT = 1.0

_NC = 2
_NS = 16
_NW = _NC * _NS
_L = 16

_N = 1048576
_PW = _N // _NW
_C = 4096
_NCHUNK = _PW // _C
_U = 4
_NT = 2

_mesh = plsc.VectorSubcoreMesh(core_axis_name="c", subcore_axis_name="s")


@functools.partial(
    pl.kernel,
    mesh=_mesh,
    out_type=jax.ShapeDtypeStruct((_NW, 16), jnp.float32),
    compiler_params=pltpu.CompilerParams(needs_layout_passes=False),
    scratch_types=[
        pltpu.VMEM((_C,), jnp.float32),
        pltpu.VMEM((_C,), jnp.int32),
        pltpu.VMEM((_C,), jnp.float32),
        pltpu.VMEM((_C,), jnp.int32),
        pltpu.SemaphoreType.DMA,
        pltpu.SemaphoreType.DMA,
        pltpu.VMEM((_NUM_SEEDS * _L,), jnp.float32),
        pltpu.VMEM((_NUM_SEEDS * _L,), jnp.float32),
        pltpu.VMEM((1, 16), jnp.float32),
    ],
)
def _seed_hist(sc_hbm, sd_hbm, out_hbm,
               sc0, sd0, sc1, sd1, sem0, sem1,
               suc_t0, suc_t1, stage_v):
    suc_t = (suc_t0, suc_t1)
    wid = lax.axis_index("s") * _NC + lax.axis_index("c")
    base = wid * _PW

    bufs = ((sc0, sd0), (sc1, sd1))
    sems = (sem0, sem1)

    def fire(k):
        slot = k % 2
        off = base + k * _C
        b = bufs[slot]
        sem = sems[slot]
        return [
            pltpu.async_copy(sc_hbm.at[pl.ds(off, _C)], b[0], sem),
            pltpu.async_copy(sd_hbm.at[pl.ds(off, _C)], b[1], sem),
        ]

    zeros16 = jnp.zeros((_L,), jnp.float32)
    iota16 = lax.iota(jnp.int32, _L)
    for t in range(_NT):
        for j in range(_NUM_SEEDS):
            suc_t[t][pl.ds(j * _L, _L)] = zeros16

    handles = {0: fire(0)}
    for k in range(_NCHUNK):
        if k + 1 < _NCHUNK:
            handles[k + 1] = fire(k + 1)
        for h in handles.pop(k):
            h.wait()
        sc_v, sd_v = bufs[k % 2]
        tab = suc_t[k % _NT]

        @plsc.parallel_loop(0, _C // _L, unroll=_U)
        def _(i):
            j = i * _L
            sc = sc_v[pl.ds(j, _L)]
            sd = sd_v[pl.ds(j, _L)]
            idx = sd * _L + iota16
            plsc.addupdate_scatter(tab, [idx], sc)

    suc = zeros16
    for b in range(_NUM_SEEDS):
        ss_b = jnp.float32(0)
        for t in range(_NT):
            ss_b = ss_b + jnp.sum(suc_t[t][pl.ds(b * _L, _L)])
        suc = jnp.where(iota16 == b, ss_b, suc)

    stage_v[0, :] = suc
    pltpu.sync_copy(stage_v, out_hbm.at[pl.ds(wid, 1)])


_DG = 8
_DR = _N // 128 // _DG


def _dense_body(fp_ref, lb_ref, sd_ref, out_ref, acc_ref):
    g = pl.program_id(0)

    @pl.when(g == 0)
    def _():
        acc_ref[...] = jnp.zeros((32, 128), jnp.float32)

    fp = fp_ref[...]
    lb = lb_ref[...]
    sd = sd_ref[...]
    d = fp - lb
    acc_ref[0, :] += jnp.sum(fp, axis=0)
    acc_ref[1, :] += jnp.sum(d * d, axis=0)
    for b in range(_NUM_SEEDS):
        acc_ref[16 + b, :] += jnp.sum((sd == b).astype(jnp.float32), axis=0)

    @pl.when(g == _DG - 1)
    def _():
        acc = acc_ref[...]
        o = lax.broadcasted_iota(jnp.int32, (1, 32), 1)
        out = jnp.zeros((1, 32), jnp.float32)
        for j in range(32):
            out = jnp.where(o == j, jnp.sum(acc[j:j + 1, :]), out)
        out_ref[...] = out


_dense = pl.pallas_call(
    _dense_body,
    grid=(_DG,),
    in_specs=[
        pl.BlockSpec((_DR, 128), lambda i: (i, 0)),
        pl.BlockSpec((_DR, 128), lambda i: (i, 0)),
        pl.BlockSpec((_DR, 128), lambda i: (i, 0)),
    ],
    out_specs=pl.BlockSpec((1, 32), lambda i: (0, 0)),
    scratch_shapes=[pltpu.VMEM((32, 128), jnp.float32)],
    out_shape=jax.ShapeDtypeStruct((1, 32), jnp.float32),
)


def _finalize_body(parts_ref, dense_ref, lam_ref, out_ref):
    parts = parts_ref[...]
    suc = jnp.sum(parts, axis=0, keepdims=True)
    cnt = dense_ref[:, 16:32]
    sum_fp = dense_ref[0, 0]
    sum_sq = dense_ref[0, 1]
    lam = lam_ref[0, 0]

    sf = (cnt - suc) / jnp.maximum(cnt, 1.0)
    idx = lax.broadcasted_iota(jnp.int32, (1, _NUM_SEEDS), 1)

    rank = jnp.zeros((1, _NUM_SEEDS), jnp.int32)
    for i in range(_NUM_SEEDS):
        sfi = jnp.sum(jnp.where(idx == i, sf, 0.0))
        gt = (sfi > sf) | ((sfi == sf) & (i < idx))
        rank = rank + gt.astype(jnp.int32)
    kk = max(1, int(_NUM_SEEDS * _CVAR_ALPHA))
    rob = jnp.sum(jnp.where(rank < kk, sf, 0.0)) * (1.0 / kk)

    inv_n = 1.0 / _N
    cost = (_COST_LLM - _COST_SLM) * (sum_fp * inv_n) + _COST_SLM
    brier = sum_sq * inv_n
    cv = rob - _CVAR_EPSILON
    lagr = lam * cv
    total = cost + lagr + _BRIER_WEIGHT * brier
    dual = -lam * cv

    o = lax.broadcasted_iota(jnp.int32, (1, 8), 1)
    out_ref[...] = jnp.where(
        o == 0, total,
        jnp.where(o == 1, cost,
                  jnp.where(o == 2, rob,
                            jnp.where(o == 3, brier,
                                      jnp.where(o == 4, lagr,
                                                jnp.where(o == 5, dual,
                                                          jnp.where(o == 6, cv, 0.0)))))))


_finalize = pl.pallas_call(
    _finalize_body,
    out_shape=jax.ShapeDtypeStruct((1, 8), jnp.float32),
)


def kernel(fallback_probs, labels, success, perturbation_seeds, lagrange_multiplier):
    seeds32 = perturbation_seeds.astype(jnp.int32)
    parts = _seed_hist(success, seeds32)
    fp2 = fallback_probs.reshape(_N // 128, 128)
    lb2 = labels.reshape(_N // 128, 128)
    sd2 = seeds32.reshape(_N // 128, 128)
    dense = _dense(fp2, lb2, sd2)
    lam2 = lagrange_multiplier.astype(jnp.float32).reshape(1, 1)
    o = _finalize(parts, dense, lam2)
    return (
        o[0, 0],
        o[0, 1],
        o[0, 2],
        o[0, 3],
        o[0, 4],
        o[0, 5],
        o[0, 6],
    )
